# baseline (speedup 1.0000x reference)
import functools

import jax
import jax.numpy as jnp
from jax import lax
from jax.experimental import pallas as pl
from jax.experimental.pallas import tpu as pltpu
from jax.experimental.pallas import tpu_sc as plsc

N = 10000
E = 160000
NF = 128
DIM = 32
NB = 64

NPAD = 10112
NC = 2
NS = 16
NW = NC * NS
EP = 163840
EP2 = EP // 2
PW2 = EP2 // NW
SUB = 128
NSUB = PW2 // SUB
IB = 24
ZR = NPAD // NS
TE = 2048


def _sc_mesh():
    return plsc.VectorSubcoreMesh(core_axis_name="c", subcore_axis_name="s",
                                  num_cores=NC)


def _gather_body(table, idxp, xs_out, idx_all, buf0, buf1, buf2,
                 g0, g1, g2, s0, s1, s2):
    cid = lax.axis_index("c")
    sid = lax.axis_index("s")
    wid = sid * NC + cid
    pltpu.sync_copy(idxp.at[pl.ds(wid * IB, IB)], idx_all)
    bufs = (buf0, buf1, buf2)
    gsem = (g0, g1, g2)
    ssem = (s0, s1, s2)
    pend_g = [None, None, None]
    pend_s = [None, None, None]
    for c in range(2):
        pend_g[c] = pltpu.async_copy(table.at[idx_all.at[c]], bufs[c],
                                     gsem[c])
    for c in range(NSUB):
        b = c % 3
        pend_g[b].wait()
        pend_s[b] = pltpu.async_copy(
            bufs[b], xs_out.at[pl.ds(wid * PW2 + c * SUB, SUB)], ssem[b])
        if c + 2 < NSUB:
            b2 = (c + 2) % 3
            if pend_s[b2] is not None:
                pend_s[b2].wait()
                pend_s[b2] = None
            pend_g[b2] = pltpu.async_copy(table.at[idx_all.at[c + 2]],
                                          bufs[b2], gsem[b2])
    for b in range(3):
        if pend_s[b] is not None:
            pend_s[b].wait()


def _sc_gather(table, idxp):
    return pl.kernel(
        _gather_body,
        mesh=_sc_mesh(),
        out_type=jax.ShapeDtypeStruct((EP2, NF), jnp.float32),
        scratch_types=[
            pltpu.VMEM((IB, SUB), jnp.int32),
            pltpu.VMEM((SUB, NF), jnp.float32),
            pltpu.VMEM((SUB, NF), jnp.float32),
            pltpu.VMEM((SUB, NF), jnp.float32),
            pltpu.SemaphoreType.DMA,
            pltpu.SemaphoreType.DMA,
            pltpu.SemaphoreType.DMA,
            pltpu.SemaphoreType.DMA,
            pltpu.SemaphoreType.DMA,
            pltpu.SemaphoreType.DMA,
        ],
    )(table, idxp)


def _scatter_body(msg, idxp, zeros_in, agg_out, idx_all, buf0, buf1,
                  shared_agg, sem0, sem1):
    cid = lax.axis_index("c")
    sid = lax.axis_index("s")
    wid = sid * NC + cid
    pltpu.sync_copy(zeros_in.at[pl.ds(sid * ZR, ZR)],
                    shared_agg.at[pl.ds(sid * ZR, ZR)])
    pltpu.sync_copy(idxp.at[pl.ds(wid * IB, IB)], idx_all)
    plsc.subcore_barrier()
    bufs = (buf0, buf1)
    sems = (sem0, sem1)
    pend = [None, None]
    for c in range(2):
        pend[c] = pltpu.async_copy(
            msg.at[pl.ds(wid * PW2 + c * SUB, SUB)], bufs[c], sems[c])
    for c in range(NSUB):
        b = c % 2
        pend[b].wait()
        pltpu.sync_copy(bufs[b], shared_agg.at[idx_all.at[c]], add=True)
        if c + 2 < NSUB:
            pend[b] = pltpu.async_copy(
                msg.at[pl.ds(wid * PW2 + (c + 2) * SUB, SUB)], bufs[b],
                sems[b])
    plsc.subcore_barrier()
    pltpu.sync_copy(shared_agg.at[pl.ds(sid * ZR, ZR)],
                    agg_out.at[pl.ds(cid * NPAD + sid * ZR, ZR)])


def _sc_scatter(msg, idxp, zeros_in):
    return pl.kernel(
        _scatter_body,
        mesh=_sc_mesh(),
        out_type=jax.ShapeDtypeStruct((2 * NPAD, NF), jnp.float32),
        scratch_types=[
            pltpu.VMEM((IB, SUB), jnp.int32),
            pltpu.VMEM((SUB, NF), jnp.float32),
            pltpu.VMEM((SUB, NF), jnp.float32),
            pltpu.VMEM_SHARED((NPAD, NF), jnp.float32),
            pltpu.SemaphoreType.DMA,
            pltpu.SemaphoreType.DMA,
        ],
    )(msg, idxp, zeros_in)


def _lin0_body(x_ref, w_ref, b_ref, o_ref):
    o = jax.nn.relu(
        jnp.dot(x_ref[...], w_ref[...], preferred_element_type=jnp.float32)
        + b_ref[...])
    o_ref[...] = jnp.concatenate(
        [o, jnp.zeros((NPAD, NF - DIM), jnp.float32)], axis=1)


def _lin0(xp, w, b):
    return pl.pallas_call(
        _lin0_body,
        out_shape=jax.ShapeDtypeStruct((NPAD, NF), jnp.float32),
    )(xp, w, b)


def _msg_body(ea_ref, xs_ref, w1_ref, b1_ref, w2f_ref, rb_ref, fold_ref,
              b2r_ref, o_ref):
    h = jax.nn.relu(
        jnp.dot(ea_ref[...], w1_ref[...], preferred_element_type=jnp.float32)
        + b1_ref[...]).astype(jnp.bfloat16)
    xs = xs_ref[:, 0:DIM]
    hgall = jnp.dot(h, w2f_ref[...], preferred_element_type=jnp.float32)
    xsrep = jnp.dot(xs.astype(jnp.bfloat16), rb_ref[...],
                    preferred_element_type=jnp.float32)
    p = (hgall * xsrep).astype(jnp.bfloat16)
    acc = jnp.dot(p, fold_ref[...], preferred_element_type=jnp.float32) \
        + jnp.dot(xs, b2r_ref[...], preferred_element_type=jnp.float32)
    o_ref[...] = jnp.concatenate(
        [acc, jnp.ones((TE, 1), jnp.float32),
         jnp.zeros((TE, NF - DIM - 1), jnp.float32)], axis=1)


def _msg(ea, xs, w1t, b1, w2f, rb, fold, b2r):
    return pl.pallas_call(
        _msg_body,
        grid=(EP2 // TE,),
        in_specs=[
            pl.BlockSpec((TE, 4), lambda i: (i, 0)),
            pl.BlockSpec((TE, NF), lambda i: (i, 0)),
            pl.BlockSpec((4, NF), lambda i: (0, 0)),
            pl.BlockSpec((1, NF), lambda i: (0, 0)),
            pl.BlockSpec((NF, DIM * DIM), lambda i: (0, 0)),
            pl.BlockSpec((DIM, DIM * DIM), lambda i: (0, 0)),
            pl.BlockSpec((DIM * DIM, DIM), lambda i: (0, 0)),
            pl.BlockSpec((DIM, DIM), lambda i: (0, 0)),
        ],
        out_specs=pl.BlockSpec((TE, NF), lambda i: (i, 0)),
        out_shape=jax.ShapeDtypeStruct((EP2, NF), jnp.float32),
    )(ea, xs, w1t, b1, w2f, rb, fold, b2r)


def _node_body(aa_ref, ab_ref, h_ref, wr_ref, cb_ref, wih_ref, bih_ref,
               whh_ref, bhh_ref, o_ref):
    agg = (aa_ref[0:NPAD, 0:DIM] + aa_ref[NPAD:2 * NPAD, 0:DIM]
           + ab_ref[0:NPAD, 0:DIM] + ab_ref[NPAD:2 * NPAD, 0:DIM])
    cnt = (aa_ref[0:NPAD, DIM:DIM + 1] + aa_ref[NPAD:2 * NPAD, DIM:DIM + 1]
           + ab_ref[0:NPAD, DIM:DIM + 1]
           + ab_ref[NPAD:2 * NPAD, DIM:DIM + 1])
    cnt = jnp.maximum(cnt, 1.0)
    h = h_ref[:, 0:DIM]
    m = jax.nn.relu(
        agg / cnt
        + jnp.dot(h, wr_ref[...], preferred_element_type=jnp.float32)
        + cb_ref[...])
    gi = jnp.dot(m, wih_ref[...], preferred_element_type=jnp.float32) \
        + bih_ref[...]
    gh = jnp.dot(h, whh_ref[...], preferred_element_type=jnp.float32) \
        + bhh_ref[...]
    r = jax.nn.sigmoid(gi[:, 0:DIM] + gh[:, 0:DIM])
    z = jax.nn.sigmoid(gi[:, DIM:2 * DIM] + gh[:, DIM:2 * DIM])
    n = jnp.tanh(gi[:, 2 * DIM:3 * DIM] + r * gh[:, 2 * DIM:3 * DIM])
    hn = (1.0 - z) * n + z * h
    o_ref[...] = jnp.concatenate(
        [hn, jnp.zeros((NPAD, NF - DIM), jnp.float32)], axis=1)


def _node(aggA, aggB, h, wr, cb, wih, bih, whh, bhh):
    return pl.pallas_call(
        _node_body,
        out_shape=jax.ShapeDtypeStruct((NPAD, NF), jnp.float32),
    )(aggA, aggB, h, wr, cb, wih, bih, whh, bhh)


def _s2s_body(hf_ref, bat_ref, wih_ref, bih_ref, whh_ref, bhh_ref,
              l1_ref, l1b_ref, l2_ref, l2b_ref, o_ref):
    hf = hf_ref[:, 0:DIM]
    iota = lax.broadcasted_iota(jnp.int32, (NPAD, NB), 1)
    mb = bat_ref[...] == iota
    mf = mb.astype(jnp.float32)
    qstar = jnp.zeros((NB, 2 * DIM), jnp.float32)
    hh = jnp.zeros((NB, DIM), jnp.float32)
    cc = jnp.zeros((NB, DIM), jnp.float32)
    for _ in range(3):
        gates = (jnp.dot(qstar, wih_ref[...],
                         preferred_element_type=jnp.float32) + bih_ref[...]
                 + jnp.dot(hh, whh_ref[...],
                           preferred_element_type=jnp.float32) + bhh_ref[...])
        gi = jax.nn.sigmoid(gates[:, 0:DIM])
        gf = jax.nn.sigmoid(gates[:, DIM:2 * DIM])
        gg = jnp.tanh(gates[:, 2 * DIM:3 * DIM])
        go = jax.nn.sigmoid(gates[:, 3 * DIM:4 * DIM])
        cc = gf * cc + gi * gg
        hh = go * jnp.tanh(cc)
        qn = jnp.dot(mf, hh, preferred_element_type=jnp.float32)
        e = jnp.sum(hf * qn, axis=1, keepdims=True)
        emax = jnp.max(jnp.where(mb, e, -1e30), axis=0, keepdims=True)
        a = jnp.exp(e - jnp.sum(mf * emax, axis=1, keepdims=True))
        asum = jnp.sum(mf * a, axis=0, keepdims=True)
        an = a / (jnp.sum(mf * asum, axis=1, keepdims=True) + 1e-16)
        rvec = lax.dot_general(mf * an, hf, (((0,), (0,)), ((), ())),
                               preferred_element_type=jnp.float32)
        qstar = jnp.concatenate([hh, rvec], axis=1)
    o1 = jax.nn.relu(
        jnp.dot(qstar, l1_ref[...], preferred_element_type=jnp.float32)
        + l1b_ref[...])
    o_ref[...] = jnp.sum(o1 * l2_ref[...], axis=1, keepdims=True) + l2b_ref[...]


def _s2s(hf, bat, wih, bih, whh, bhh, l1, l1b, l2, l2b):
    return pl.pallas_call(
        _s2s_body,
        out_shape=jax.ShapeDtypeStruct((NB, 1), jnp.float32),
    )(hf, bat, wih, bih, whh, bhh, l1, l1b, l2, l2b)


def _idx_blocks(v):
    b = v.reshape(NW, NSUB, SUB)
    b = jnp.concatenate(
        [b, jnp.zeros((NW, IB - NSUB, SUB), jnp.int32)], axis=1)
    return b.reshape(NW * IB, SUB)


def kernel(x, edge_attr, edge_index, batch, lin0_w, lin0_b, i_w1, i_b1,
           i_w2, i_b2, conv_root_w, conv_bias, gru_w_ih, gru_w_hh,
           gru_b_ih, gru_b_hh, ls_w_ih, ls_w_hh, ls_b_ih, ls_b_hh,
           lin1_w, lin1_b, lin2_w, lin2_b):
    f32 = jnp.float32
    src = edge_index[0]
    dst = edge_index[1]
    src_p = jnp.concatenate([src, jnp.zeros((EP - E,), jnp.int32)])
    dst_p = jnp.concatenate([dst, jnp.full((EP - E,), N, jnp.int32)])
    ea_p = jnp.concatenate([edge_attr, jnp.zeros((EP - E, 4), f32)], axis=0)
    srcH = src_p.reshape(NW, 2, PW2)
    dstH = dst_p.reshape(NW, 2, PW2)
    eaH = ea_p.reshape(NW, 2, PW2, 4)
    src2dA = _idx_blocks(srcH[:, 0])
    src2dB = _idx_blocks(srcH[:, 1])
    dst2dA = _idx_blocks(dstH[:, 0])
    dst2dB = _idx_blocks(dstH[:, 1])
    eaA = eaH[:, 0].reshape(EP2, 4)
    eaB = eaH[:, 1].reshape(EP2, 4)
    x_pad = jnp.concatenate([x, jnp.zeros((NPAD - N, NF), f32)], axis=0)
    bat2d = jnp.concatenate(
        [batch, jnp.full((NPAD - N,), NB, jnp.int32)]).reshape(NPAD, 1)
    zeros_in = jnp.zeros((NPAD, NF), f32)

    w2f = i_w2.reshape(DIM, DIM, NF).transpose(2, 0, 1).reshape(
        NF, DIM * DIM).astype(jnp.bfloat16)
    rb = jnp.repeat(jnp.eye(DIM, dtype=f32), DIM,
                    axis=1).astype(jnp.bfloat16)
    fold = jnp.tile(jnp.eye(DIM, dtype=f32),
                    (DIM, 1)).astype(jnp.bfloat16)
    b2r = i_b2.reshape(DIM, DIM)
    w1t = i_w1.T
    b1 = i_b1.reshape(1, NF)

    state = _lin0(x_pad, lin0_w.T, lin0_b.reshape(1, DIM))
    for _ in range(3):
        xsA = _sc_gather(state, src2dA)
        xsB = _sc_gather(state, src2dB)
        msgA = _msg(eaA, xsA, w1t, b1, w2f, rb, fold, b2r)
        aggA = _sc_scatter(msgA, dst2dA, zeros_in)
        msgB = _msg(eaB, xsB, w1t, b1, w2f, rb, fold, b2r)
        aggB = _sc_scatter(msgB, dst2dB, zeros_in)
        state = _node(aggA, aggB, state, conv_root_w,
                      conv_bias.reshape(1, DIM),
                      gru_w_ih.T, gru_b_ih.reshape(1, 3 * DIM),
                      gru_w_hh.T, gru_b_hh.reshape(1, 3 * DIM))
    o = _s2s(state, bat2d,
             ls_w_ih.T, ls_b_ih.reshape(1, 4 * DIM),
             ls_w_hh.T, ls_b_hh.reshape(1, 4 * DIM),
             lin1_w.T, lin1_b.reshape(1, DIM),
             lin2_w, lin2_b.reshape(1, 1))
    return o.reshape(-1)


# speedup vs baseline: 1.1280x; 1.1280x over previous
import functools

import jax
import jax.numpy as jnp
from jax import lax
from jax.experimental import pallas as pl
from jax.experimental.pallas import tpu as pltpu
from jax.experimental.pallas import tpu_sc as plsc

N = 10000
E = 160000
NF = 128
DIM = 32
NB = 64

NPAD = 10112
NC = 2
NS = 16
NW = NC * NS
EP = 163840
EP2 = EP // 2
PW2 = EP2 // NW
SUB = 128
NSUB = PW2 // SUB
IB = 24
ZR = NPAD // NS
TE = 2048


def _sc_mesh():
    return plsc.VectorSubcoreMesh(core_axis_name="c", subcore_axis_name="s",
                                  num_cores=NC)


def _gather_body(table, idxp, xs_out, idx_all, buf0, buf1, buf2,
                 g0, g1, g2, s0, s1, s2):
    cid = lax.axis_index("c")
    sid = lax.axis_index("s")
    wid = sid * NC + cid
    pltpu.sync_copy(idxp.at[pl.ds(wid * IB, IB)], idx_all)
    bufs = (buf0, buf1, buf2)
    gsem = (g0, g1, g2)
    ssem = (s0, s1, s2)
    pend_g = [None, None, None]
    pend_s = [None, None, None]
    for c in range(2):
        pend_g[c] = pltpu.async_copy(table.at[idx_all.at[c]], bufs[c],
                                     gsem[c])
    for c in range(NSUB):
        b = c % 3
        pend_g[b].wait()
        pend_s[b] = pltpu.async_copy(
            bufs[b], xs_out.at[pl.ds(wid * PW2 + c * SUB, SUB)], ssem[b])
        if c + 2 < NSUB:
            b2 = (c + 2) % 3
            if pend_s[b2] is not None:
                pend_s[b2].wait()
                pend_s[b2] = None
            pend_g[b2] = pltpu.async_copy(table.at[idx_all.at[c + 2]],
                                          bufs[b2], gsem[b2])
    for b in range(3):
        if pend_s[b] is not None:
            pend_s[b].wait()


def _sc_gather(table, idxp):
    return pl.kernel(
        _gather_body,
        mesh=_sc_mesh(),
        out_type=jax.ShapeDtypeStruct((EP2, NF), jnp.float32),
        scratch_types=[
            pltpu.VMEM((IB, SUB), jnp.int32),
            pltpu.VMEM((SUB, NF), jnp.float32),
            pltpu.VMEM((SUB, NF), jnp.float32),
            pltpu.VMEM((SUB, NF), jnp.float32),
            pltpu.SemaphoreType.DMA,
            pltpu.SemaphoreType.DMA,
            pltpu.SemaphoreType.DMA,
            pltpu.SemaphoreType.DMA,
            pltpu.SemaphoreType.DMA,
            pltpu.SemaphoreType.DMA,
        ],
    )(table, idxp)


def _scatter_body(msg, idxp, zeros_in, agg_out, idx_all, buf0, buf1,
                  shared_agg, sem0, sem1):
    cid = lax.axis_index("c")
    sid = lax.axis_index("s")
    wid = sid * NC + cid
    pltpu.sync_copy(zeros_in.at[pl.ds(sid * ZR, ZR)],
                    shared_agg.at[pl.ds(sid * ZR, ZR)])
    pltpu.sync_copy(idxp.at[pl.ds(wid * IB, IB)], idx_all)
    plsc.subcore_barrier()
    bufs = (buf0, buf1)
    sems = (sem0, sem1)
    pend = [None, None]
    for c in range(2):
        pend[c] = pltpu.async_copy(
            msg.at[pl.ds(wid * PW2 + c * SUB, SUB)], bufs[c], sems[c])
    for c in range(NSUB):
        b = c % 2
        pend[b].wait()
        pltpu.sync_copy(bufs[b], shared_agg.at[idx_all.at[c]], add=True)
        if c + 2 < NSUB:
            pend[b] = pltpu.async_copy(
                msg.at[pl.ds(wid * PW2 + (c + 2) * SUB, SUB)], bufs[b],
                sems[b])
    plsc.subcore_barrier()
    pltpu.sync_copy(shared_agg.at[pl.ds(sid * ZR, ZR)],
                    agg_out.at[pl.ds(cid * NPAD + sid * ZR, ZR)])


def _sc_scatter(msg, idxp, zeros_in):
    return pl.kernel(
        _scatter_body,
        mesh=_sc_mesh(),
        out_type=jax.ShapeDtypeStruct((2 * NPAD, NF), jnp.float32),
        scratch_types=[
            pltpu.VMEM((IB, SUB), jnp.int32),
            pltpu.VMEM((SUB, NF), jnp.float32),
            pltpu.VMEM((SUB, NF), jnp.float32),
            pltpu.VMEM_SHARED((NPAD, NF), jnp.float32),
            pltpu.SemaphoreType.DMA,
            pltpu.SemaphoreType.DMA,
        ],
    )(msg, idxp, zeros_in)


def _lin0_body(x_ref, w_ref, b_ref, o_ref):
    o = jax.nn.relu(
        jnp.dot(x_ref[...], w_ref[...], preferred_element_type=jnp.float32)
        + b_ref[...])
    o_ref[...] = jnp.concatenate(
        [o, jnp.zeros((NPAD, NF - DIM), jnp.float32)], axis=1)


def _lin0(xp, w, b):
    return pl.pallas_call(
        _lin0_body,
        out_shape=jax.ShapeDtypeStruct((NPAD, NF), jnp.float32),
    )(xp, w, b)


def _msg_body(ea_ref, xs_ref, w1_ref, b1_ref, w2f_ref, rb_ref, fold_ref,
              b2r_ref, o_ref):
    h = jax.nn.relu(
        jnp.dot(ea_ref[...], w1_ref[...], preferred_element_type=jnp.float32)
        + b1_ref[...]).astype(jnp.bfloat16)
    xs = xs_ref[:, 0:DIM]
    hgall = jnp.dot(h, w2f_ref[...], preferred_element_type=jnp.float32)
    xsrep = jnp.dot(xs.astype(jnp.bfloat16), rb_ref[...],
                    preferred_element_type=jnp.float32)
    p = hgall * xsrep
    s = DIM * DIM
    while s > DIM:
        s = s // 2
        p = p[:, 0:s] + p[:, s:2 * s]
    acc = p + jnp.dot(xs, b2r_ref[...], preferred_element_type=jnp.float32)
    o_ref[...] = jnp.concatenate(
        [acc, jnp.ones((TE, 1), jnp.float32),
         jnp.zeros((TE, NF - DIM - 1), jnp.float32)], axis=1)


def _msg(ea, xs, w1t, b1, w2f, rb, fold, b2r):
    return pl.pallas_call(
        _msg_body,
        grid=(EP2 // TE,),
        in_specs=[
            pl.BlockSpec((TE, 4), lambda i: (i, 0)),
            pl.BlockSpec((TE, NF), lambda i: (i, 0)),
            pl.BlockSpec((4, NF), lambda i: (0, 0)),
            pl.BlockSpec((1, NF), lambda i: (0, 0)),
            pl.BlockSpec((NF, DIM * DIM), lambda i: (0, 0)),
            pl.BlockSpec((DIM, DIM * DIM), lambda i: (0, 0)),
            pl.BlockSpec((DIM * DIM, DIM), lambda i: (0, 0)),
            pl.BlockSpec((DIM, DIM), lambda i: (0, 0)),
        ],
        out_specs=pl.BlockSpec((TE, NF), lambda i: (i, 0)),
        out_shape=jax.ShapeDtypeStruct((EP2, NF), jnp.float32),
    )(ea, xs, w1t, b1, w2f, rb, fold, b2r)


def _node_body(aa_ref, ab_ref, h_ref, wr_ref, cb_ref, wih_ref, bih_ref,
               whh_ref, bhh_ref, o_ref):
    agg = (aa_ref[0:NPAD, 0:DIM] + aa_ref[NPAD:2 * NPAD, 0:DIM]
           + ab_ref[0:NPAD, 0:DIM] + ab_ref[NPAD:2 * NPAD, 0:DIM])
    cnt = (aa_ref[0:NPAD, DIM:DIM + 1] + aa_ref[NPAD:2 * NPAD, DIM:DIM + 1]
           + ab_ref[0:NPAD, DIM:DIM + 1]
           + ab_ref[NPAD:2 * NPAD, DIM:DIM + 1])
    cnt = jnp.maximum(cnt, 1.0)
    h = h_ref[:, 0:DIM]
    m = jax.nn.relu(
        agg / cnt
        + jnp.dot(h, wr_ref[...], preferred_element_type=jnp.float32)
        + cb_ref[...])
    gi = jnp.dot(m, wih_ref[...], preferred_element_type=jnp.float32) \
        + bih_ref[...]
    gh = jnp.dot(h, whh_ref[...], preferred_element_type=jnp.float32) \
        + bhh_ref[...]
    r = jax.nn.sigmoid(gi[:, 0:DIM] + gh[:, 0:DIM])
    z = jax.nn.sigmoid(gi[:, DIM:2 * DIM] + gh[:, DIM:2 * DIM])
    n = jnp.tanh(gi[:, 2 * DIM:3 * DIM] + r * gh[:, 2 * DIM:3 * DIM])
    hn = (1.0 - z) * n + z * h
    o_ref[...] = jnp.concatenate(
        [hn, jnp.zeros((NPAD, NF - DIM), jnp.float32)], axis=1)


def _node(aggA, aggB, h, wr, cb, wih, bih, whh, bhh):
    return pl.pallas_call(
        _node_body,
        out_shape=jax.ShapeDtypeStruct((NPAD, NF), jnp.float32),
    )(aggA, aggB, h, wr, cb, wih, bih, whh, bhh)


def _s2s_body(hf_ref, bat_ref, wih_ref, bih_ref, whh_ref, bhh_ref,
              l1_ref, l1b_ref, l2_ref, l2b_ref, o_ref):
    hf = hf_ref[:, 0:DIM]
    iota = lax.broadcasted_iota(jnp.int32, (NPAD, NB), 1)
    mb = bat_ref[...] == iota
    mf = mb.astype(jnp.float32)
    qstar = jnp.zeros((NB, 2 * DIM), jnp.float32)
    hh = jnp.zeros((NB, DIM), jnp.float32)
    cc = jnp.zeros((NB, DIM), jnp.float32)
    for _ in range(3):
        gates = (jnp.dot(qstar, wih_ref[...],
                         preferred_element_type=jnp.float32) + bih_ref[...]
                 + jnp.dot(hh, whh_ref[...],
                           preferred_element_type=jnp.float32) + bhh_ref[...])
        gi = jax.nn.sigmoid(gates[:, 0:DIM])
        gf = jax.nn.sigmoid(gates[:, DIM:2 * DIM])
        gg = jnp.tanh(gates[:, 2 * DIM:3 * DIM])
        go = jax.nn.sigmoid(gates[:, 3 * DIM:4 * DIM])
        cc = gf * cc + gi * gg
        hh = go * jnp.tanh(cc)
        qn = jnp.dot(mf, hh, preferred_element_type=jnp.float32)
        e = jnp.sum(hf * qn, axis=1, keepdims=True)
        emax = jnp.max(jnp.where(mb, e, -1e30), axis=0, keepdims=True)
        a = jnp.exp(e - jnp.sum(mf * emax, axis=1, keepdims=True))
        asum = jnp.sum(mf * a, axis=0, keepdims=True)
        an = a / (jnp.sum(mf * asum, axis=1, keepdims=True) + 1e-16)
        rvec = lax.dot_general(mf * an, hf, (((0,), (0,)), ((), ())),
                               preferred_element_type=jnp.float32)
        qstar = jnp.concatenate([hh, rvec], axis=1)
    o1 = jax.nn.relu(
        jnp.dot(qstar, l1_ref[...], preferred_element_type=jnp.float32)
        + l1b_ref[...])
    o_ref[...] = jnp.sum(o1 * l2_ref[...], axis=1, keepdims=True) + l2b_ref[...]


def _s2s(hf, bat, wih, bih, whh, bhh, l1, l1b, l2, l2b):
    return pl.pallas_call(
        _s2s_body,
        out_shape=jax.ShapeDtypeStruct((NB, 1), jnp.float32),
    )(hf, bat, wih, bih, whh, bhh, l1, l1b, l2, l2b)


def _idx_blocks(v):
    b = v.reshape(NW, NSUB, SUB)
    b = jnp.concatenate(
        [b, jnp.zeros((NW, IB - NSUB, SUB), jnp.int32)], axis=1)
    return b.reshape(NW * IB, SUB)


def kernel(x, edge_attr, edge_index, batch, lin0_w, lin0_b, i_w1, i_b1,
           i_w2, i_b2, conv_root_w, conv_bias, gru_w_ih, gru_w_hh,
           gru_b_ih, gru_b_hh, ls_w_ih, ls_w_hh, ls_b_ih, ls_b_hh,
           lin1_w, lin1_b, lin2_w, lin2_b):
    f32 = jnp.float32
    src = edge_index[0]
    dst = edge_index[1]
    src_p = jnp.concatenate([src, jnp.zeros((EP - E,), jnp.int32)])
    dst_p = jnp.concatenate([dst, jnp.full((EP - E,), N, jnp.int32)])
    ea_p = jnp.concatenate([edge_attr, jnp.zeros((EP - E, 4), f32)], axis=0)
    srcH = src_p.reshape(NW, 2, PW2)
    dstH = dst_p.reshape(NW, 2, PW2)
    eaH = ea_p.reshape(NW, 2, PW2, 4)
    src2dA = _idx_blocks(srcH[:, 0])
    src2dB = _idx_blocks(srcH[:, 1])
    dst2dA = _idx_blocks(dstH[:, 0])
    dst2dB = _idx_blocks(dstH[:, 1])
    eaA = eaH[:, 0].reshape(EP2, 4)
    eaB = eaH[:, 1].reshape(EP2, 4)
    x_pad = jnp.concatenate([x, jnp.zeros((NPAD - N, NF), f32)], axis=0)
    bat2d = jnp.concatenate(
        [batch, jnp.full((NPAD - N,), NB, jnp.int32)]).reshape(NPAD, 1)
    zeros_in = jnp.zeros((NPAD, NF), f32)

    w2f = i_w2.reshape(DIM, DIM, NF).transpose(2, 0, 1).reshape(
        NF, DIM * DIM).astype(jnp.bfloat16)
    rb = jnp.repeat(jnp.eye(DIM, dtype=f32), DIM,
                    axis=1).astype(jnp.bfloat16)
    fold = jnp.tile(jnp.eye(DIM, dtype=f32),
                    (DIM, 1)).astype(jnp.bfloat16)
    b2r = i_b2.reshape(DIM, DIM)
    w1t = i_w1.T
    b1 = i_b1.reshape(1, NF)

    state = _lin0(x_pad, lin0_w.T, lin0_b.reshape(1, DIM))
    for _ in range(3):
        xsA = _sc_gather(state, src2dA)
        xsB = _sc_gather(state, src2dB)
        msgA = _msg(eaA, xsA, w1t, b1, w2f, rb, fold, b2r)
        aggA = _sc_scatter(msgA, dst2dA, zeros_in)
        msgB = _msg(eaB, xsB, w1t, b1, w2f, rb, fold, b2r)
        aggB = _sc_scatter(msgB, dst2dB, zeros_in)
        state = _node(aggA, aggB, state, conv_root_w,
                      conv_bias.reshape(1, DIM),
                      gru_w_ih.T, gru_b_ih.reshape(1, 3 * DIM),
                      gru_w_hh.T, gru_b_hh.reshape(1, 3 * DIM))
    o = _s2s(state, bat2d,
             ls_w_ih.T, ls_b_ih.reshape(1, 4 * DIM),
             ls_w_hh.T, ls_b_hh.reshape(1, 4 * DIM),
             lin1_w.T, lin1_b.reshape(1, DIM),
             lin2_w, lin2_b.reshape(1, 1))
    return o.reshape(-1)
